# baseline (speedup 1.0000x reference)
import jax
import jax.numpy as jnp
from jax import lax
from jax.experimental import pallas as pl
from jax.experimental.pallas import tpu as pltpu
from jax.experimental.pallas import tpu_sc as plsc

BATCH = 128
NF = 32768
NC = 2
NS = 16
NW = NC * NS
ROWS_PER_W = BATCH // NW
CHUNK = 16384
NCHUNK = NF // CHUNK
LANES = 16
UNROLL = 8
ITERS = CHUNK // (LANES * UNROLL)

_F32_BIG = 3.4e38


def _closest_body(in_hbm, prev_hbm, out_hbm, buf0, buf1, pv_all, res_buf,
                  sem0, sem1):
    wid = lax.axis_index("s") * NC + lax.axis_index("c")

    pltpu.sync_copy(prev_hbm, pv_all)

    def start(row, c, buf, sem):
        return pltpu.async_copy(
            in_hbm.at[row, pl.ds(c * CHUNK, CHUNK)], buf, sem
        )

    start(wid * ROWS_PER_W, 0, buf0, sem0)

    def scan_chunk(buf, pv, mind, minv):
        def body(i, carry):
            acc = list(carry)
            for k in range(UNROLL):
                v = buf[pl.ds(i * (LANES * UNROLL) + k * LANES, LANES)]
                d = jnp.abs(v - pv)
                md, mv = acc[k], acc[UNROLL + k]
                pred = d < md
                acc[k] = jnp.where(pred, d, md)
                acc[UNROLL + k] = jnp.where(pred, v, mv)
            return tuple(acc)

        res = lax.fori_loop(0, ITERS, body, tuple(mind) + tuple(minv))
        return list(res[:UNROLL]), list(res[UNROLL:])

    def row_body(r, carry):
        row = wid * ROWS_PER_W + r
        pv = plsc.load_gather(pv_all, [jnp.full((LANES,), row, jnp.int32)])

        d1 = start(row, 1, buf1, sem1)
        pltpu.make_async_copy(
            in_hbm.at[row, pl.ds(0, CHUNK)], buf0, sem0
        ).wait()

        mind = [jnp.full((LANES,), _F32_BIG, jnp.float32)] * UNROLL
        minv = [jnp.zeros((LANES,), jnp.float32)] * UNROLL
        mind, minv = scan_chunk(buf0, pv, mind, minv)

        @pl.when(r < ROWS_PER_W - 1)
        def _():
            start(row + 1, 0, buf0, sem0)

        d1.wait()
        mind, minv = scan_chunk(buf1, pv, mind, minv)

        n = UNROLL
        while n > 1:
            n //= 2
            for k in range(n):
                pred = mind[k + n] < mind[k]
                mind[k] = jnp.where(pred, mind[k + n], mind[k])
                minv[k] = jnp.where(pred, minv[k + n], minv[k])
        _, vs = plsc.sort_key_val(mind[0], minv[0])
        res_buf[...] = vs
        pltpu.sync_copy(res_buf, out_hbm.at[row])
        return carry

    lax.fori_loop(0, ROWS_PER_W, row_body, jnp.int32(0))


@jax.jit
def _closest(inp, prev_flat):
    mesh = plsc.VectorSubcoreMesh(core_axis_name="c", subcore_axis_name="s")
    f = pl.kernel(
        _closest_body,
        out_type=jax.ShapeDtypeStruct((BATCH, LANES), jnp.float32),
        mesh=mesh,
        compiler_params=pltpu.CompilerParams(needs_layout_passes=False),
        scratch_types=[
            pltpu.VMEM((CHUNK,), jnp.float32),
            pltpu.VMEM((CHUNK,), jnp.float32),
            pltpu.VMEM((BATCH,), jnp.float32),
            pltpu.VMEM((LANES,), jnp.float32),
            pltpu.SemaphoreType.DMA,
            pltpu.SemaphoreType.DMA,
        ],
    )
    return f(inp, prev_flat)


def kernel(input, prev_output):
    out = _closest(input, prev_output.reshape(BATCH))
    return out[:, :1]
